# Initial kernel scaffold; baseline (speedup 1.0000x reference)
import jax
import jax.numpy as jnp
from jax.experimental import pallas as pl


def kernel(temporal_id, zone_id, temporal_table, zone_table):
    raise NotImplementedError("write your pallas kernel here")


# speedup vs baseline: 2.5600x; 2.5600x over previous
import functools

import jax
import jax.numpy as jnp
from jax import lax
from jax.experimental import pallas as pl
from jax.experimental.pallas import tpu as pltpu
from jax.experimental.pallas import tpu_sc as plsc

TEMP_VOCAB = 25
ZONE_VOCAB = 6067
D = 256
ROWS = 4096 * 200
NUM_WORKERS = 32
PER_W = ROWS // NUM_WORKERS
B = 128
NBATCH = PER_W // B
LANES = 16


def _make_kernel():
    mesh = plsc.VectorSubcoreMesh(core_axis_name="c", subcore_axis_name="s")

    @functools.partial(
        pl.kernel,
        mesh=mesh,
        out_type=jax.ShapeDtypeStruct((ROWS, D), jnp.float32),
        scratch_types=[
            pltpu.VMEM((B,), jnp.int32),
            pltpu.VMEM((B,), jnp.int32),
            pltpu.VMEM((B, D), jnp.float32),
            pltpu.VMEM((B, D), jnp.float32),
            pltpu.SemaphoreType.DMA,
            pltpu.SemaphoreType.DMA,
        ],
    )
    def k(tid_hbm, zid_hbm, ttab_hbm, ztab_hbm, out_hbm,
          tid_v, zid_v, tbuf, zbuf, sem_t, sem_z):
        wid = lax.axis_index("s") * 2 + lax.axis_index("c")
        base = wid * PER_W

        def batch_body(i, carry):
            off = base + i * B
            pltpu.sync_copy(tid_hbm.at[pl.ds(off, B)], tid_v)
            pltpu.sync_copy(zid_hbm.at[pl.ds(off, B)], zid_v)
            ct = pltpu.async_copy(ttab_hbm.at[tid_v], tbuf, sem_t)
            cz = pltpu.async_copy(ztab_hbm.at[zid_v], zbuf, sem_z)
            ct.wait()
            cz.wait()

            def row_body(r, c):
                for j in range(D // LANES):
                    sl = pl.ds(j * LANES, LANES)
                    zbuf[r, sl] = zbuf[r, sl] + tbuf[r, sl]
                return c

            lax.fori_loop(0, B, row_body, 0)
            pltpu.sync_copy(zbuf, out_hbm.at[pl.ds(off, B)])
            return carry

        lax.fori_loop(0, NBATCH, batch_body, 0)

    return k


_kernel = _make_kernel()


def kernel(temporal_id, zone_id, temporal_table, zone_table):
    tid = temporal_id.reshape(-1).astype(jnp.int32)
    zid = zone_id.reshape(-1).astype(jnp.int32)
    out = _kernel(tid, zid, temporal_table, zone_table)
    return out.reshape(temporal_id.shape + (D,))


# speedup vs baseline: 2.9708x; 1.1605x over previous
import functools

import jax
import jax.numpy as jnp
from jax import lax
from jax.experimental import pallas as pl
from jax.experimental.pallas import tpu as pltpu
from jax.experimental.pallas import tpu_sc as plsc

TEMP_VOCAB = 25
D = 256
ROWS = 4096 * 200
NUM_WORKERS = 32
PER_W = ROWS // NUM_WORKERS
B = 128
NBATCH = PER_W // B
LANES = 16


def _make_kernel():
    mesh = plsc.VectorSubcoreMesh(core_axis_name="c", subcore_axis_name="s")

    @functools.partial(
        pl.kernel,
        mesh=mesh,
        out_type=jax.ShapeDtypeStruct((ROWS, D), jnp.float32),
        scratch_types=[
            pltpu.VMEM((2, B), jnp.int32),
            pltpu.VMEM((2, B), jnp.int32),
            pltpu.VMEM((2, B, D), jnp.float32),
            pltpu.VMEM((TEMP_VOCAB, D), jnp.float32),
            pltpu.SemaphoreType.DMA,
            pltpu.SemaphoreType.DMA,
        ],
    )
    def k(tid_hbm, zid_hbm, ttab_hbm, ztab_hbm, out_hbm,
          tidb, zidb, zbuf, ttab_v, semz0, semz1):
        sid = lax.axis_index("s")
        wid = sid * 2 + lax.axis_index("c")
        base = wid * PER_W
        semz = (semz0, semz1)

        pltpu.sync_copy(ttab_hbm, ttab_v)

        def start_batch(i, p):
            off = base + i * B
            pltpu.sync_copy(tid_hbm.at[pl.ds(off, B)], tidb.at[p])
            pltpu.sync_copy(zid_hbm.at[pl.ds(off, B)], zidb.at[p])
            pltpu.async_copy(ztab_hbm.at[zidb.at[p]], zbuf.at[p], semz[p])

        def wait_batch(p):
            pltpu.make_async_copy(ztab_hbm.at[zidb.at[p]], zbuf.at[p], semz[p]).wait()

        start_batch(0, 0)

        def pair_body(gp, carry):
            for p in (0, 1):
                i = gp * 2 + p

                @pl.when(i + 1 < NBATCH)
                def _():
                    start_batch(i + 1, 1 - p)

                wait_batch(p)
                zb = zbuf.at[p]

                def group_body(g, c):
                    tvec = tidb[p, pl.ds(g * LANES, LANES)]
                    for rr in range(LANES):
                        tid = tvec[rr]
                        r = g * LANES + rr
                        for j in range(D // LANES):
                            sl = pl.ds(j * LANES, LANES)
                            zb[r, sl] = zb[r, sl] + ttab_v[tid, sl]
                    return c

                lax.fori_loop(0, B // LANES, group_body, 0)
                pltpu.sync_copy(zb, out_hbm.at[pl.ds(base + i * B, B)])
            return carry

        lax.fori_loop(0, NBATCH // 2, pair_body, 0)

    return k


_kernel = _make_kernel()


def kernel(temporal_id, zone_id, temporal_table, zone_table):
    tid = temporal_id.reshape(-1).astype(jnp.int32)
    zid = zone_id.reshape(-1).astype(jnp.int32)
    out = _kernel(tid, zid, temporal_table, zone_table)
    return out.reshape(temporal_id.shape + (D,))


# speedup vs baseline: 3.7008x; 1.2457x over previous
import functools

import jax
import jax.numpy as jnp
from jax import lax
from jax.experimental import pallas as pl
from jax.experimental.pallas import tpu as pltpu
from jax.experimental.pallas import tpu_sc as plsc

TEMP_VOCAB = 25
D = 256
ROWS = 4096 * 200
NUM_WORKERS = 32
PER_W = ROWS // NUM_WORKERS
B = 128
NBATCH = PER_W // B
LANES = 16


def _make_kernel():
    mesh = plsc.VectorSubcoreMesh(core_axis_name="c", subcore_axis_name="s")

    @functools.partial(
        pl.kernel,
        mesh=mesh,
        out_type=jax.ShapeDtypeStruct((ROWS, D), jnp.float32),
        scratch_types=[
            pltpu.VMEM((2, B), jnp.int32),
            pltpu.VMEM((2, B), jnp.int32),
            pltpu.VMEM((2, B, D), jnp.float32),
            pltpu.VMEM((TEMP_VOCAB, D), jnp.float32),
            pltpu.SemaphoreType.DMA,
            pltpu.SemaphoreType.DMA,
            pltpu.SemaphoreType.DMA,
            pltpu.SemaphoreType.DMA,
        ],
    )
    def k(tid_hbm, zid_hbm, ttab_hbm, ztab_hbm, out_hbm,
          tidb, zidb, zbuf, ttab_v, semz0, semz1, semo0, semo1):
        sid = lax.axis_index("s")
        wid = sid * 2 + lax.axis_index("c")
        base = wid * PER_W
        semz = (semz0, semz1)
        semo = (semo0, semo1)

        pltpu.sync_copy(ttab_hbm, ttab_v)

        def start_batch(i, p):
            off = base + i * B
            pltpu.sync_copy(tid_hbm.at[pl.ds(off, B)], tidb.at[p])
            pltpu.sync_copy(zid_hbm.at[pl.ds(off, B)], zidb.at[p])
            pltpu.async_copy(ztab_hbm.at[zidb.at[p]], zbuf.at[p], semz[p])

        def wait_batch(p):
            pltpu.make_async_copy(ztab_hbm.at[zidb.at[p]], zbuf.at[p], semz[p]).wait()

        def wait_outwrite(i, p):
            pltpu.make_async_copy(
                zbuf.at[p], out_hbm.at[pl.ds(base + i * B, B)], semo[p]).wait()

        start_batch(0, 0)

        def pair_body(gp, carry):
            for p in (0, 1):
                i = gp * 2 + p

                @pl.when(i + 1 < NBATCH)
                def _():
                    @pl.when(i >= 1)
                    def _():
                        wait_outwrite(i - 1, 1 - p)

                    start_batch(i + 1, 1 - p)

                wait_batch(p)
                zb = zbuf.at[p]

                def group_body(g, c):
                    tvec = tidb[p, pl.ds(g * LANES, LANES)]
                    for rr in range(LANES):
                        tid = tvec[rr]
                        r = g * LANES + rr
                        for j in range(D // LANES):
                            sl = pl.ds(j * LANES, LANES)
                            plsc.addupdate(zb.at[r, sl], ttab_v[tid, sl])
                    return c

                lax.fori_loop(0, B // LANES, group_body, 0)
                pltpu.async_copy(zb, out_hbm.at[pl.ds(base + i * B, B)], semo[p])
            return carry

        lax.fori_loop(0, NBATCH // 2, pair_body, 0)
        wait_outwrite(NBATCH - 2, 0)
        wait_outwrite(NBATCH - 1, 1)

    return k


_kernel = _make_kernel()


def kernel(temporal_id, zone_id, temporal_table, zone_table):
    tid = temporal_id.reshape(-1).astype(jnp.int32)
    zid = zone_id.reshape(-1).astype(jnp.int32)
    out = _kernel(tid, zid, temporal_table, zone_table)
    return out.reshape(temporal_id.shape + (D,))
